# Initial kernel scaffold; baseline (speedup 1.0000x reference)
import jax
import jax.numpy as jnp
from jax.experimental import pallas as pl


def kernel(input_tensor, adjacency_matrix, node_degree, W, b):
    raise NotImplementedError("write your pallas kernel here")


# speedup vs baseline: 6.5120x; 6.5120x over previous
import functools

import jax
import jax.numpy as jnp
from jax.experimental import pallas as pl
from jax.experimental.pallas import tpu as pltpu

BM = 1000
BK = 2048


def _gcn_body(n, a_ref, xp_ref, deg_ref, wt_ref, b_ref, out_ref, acc_ref):
    i = pl.program_id(0)
    k = pl.program_id(1)
    nk = pl.num_programs(1)

    @pl.when(k == 0)
    def _():
        acc_ref[...] = jnp.zeros_like(acc_ref)

    xk = xp_ref[pl.ds(k * BK, BK), :]

    @pl.when(k < nk - 1)
    def _():
        acc_ref[...] += jnp.dot(a_ref[...], xk,
                                preferred_element_type=jnp.float32)

    @pl.when(k == nk - 1)
    def _():
        col = k * BK + jax.lax.broadcasted_iota(jnp.int32, (1, BK), 1)
        a = jnp.where(col < n, a_ref[...], 0.0)
        acc = acc_ref[...] + jnp.dot(a, xk,
                                     preferred_element_type=jnp.float32)
        xr = xp_ref[pl.ds(i * BM, BM), :]
        inv = 1.0 / deg_ref[...]
        pool = inv * (acc + xr) + xr
        out = jnp.dot(pool, wt_ref[...], preferred_element_type=jnp.float32)
        out_ref[...] = jnp.maximum(out + b_ref[...], 0.0)


@jax.jit
def kernel(input_tensor, adjacency_matrix, node_degree, W, b):
    n, d_in = input_tensor.shape
    d_out = W.shape[0]
    gm = n // BM
    gk = pl.cdiv(n, BK)
    padn = gk * BK

    xp = jnp.pad(input_tensor, ((0, padn - n), (0, 0)))
    wt = W.T
    b2 = b.reshape(1, d_out)

    return pl.pallas_call(
        functools.partial(_gcn_body, n),
        grid=(gm, gk),
        in_specs=[
            pl.BlockSpec((BM, BK), lambda i, k: (i, k)),
            pl.BlockSpec((padn, d_in), lambda i, k: (0, 0)),
            pl.BlockSpec((BM, 1), lambda i, k: (i, 0)),
            pl.BlockSpec((d_in, d_out), lambda i, k: (0, 0)),
            pl.BlockSpec((1, d_out), lambda i, k: (0, 0)),
        ],
        out_specs=pl.BlockSpec((BM, d_out), lambda i, k: (i, 0)),
        out_shape=jax.ShapeDtypeStruct((n, d_out), jnp.float32),
        scratch_shapes=[pltpu.VMEM((BM, d_out), jnp.float32)],
        compiler_params=pltpu.CompilerParams(
            dimension_semantics=("parallel", "arbitrary")),
    )(adjacency_matrix, xp, node_degree, wt, b2)


# speedup vs baseline: 6.5194x; 1.0011x over previous
import functools

import jax
import jax.numpy as jnp
from jax.experimental import pallas as pl
from jax.experimental.pallas import tpu as pltpu

BM = 1000
BK = 2048


def _gcn_body(n, a_ref, xp_ref, deg_ref, wt_ref, b_ref, out_ref, acc_ref):
    i = pl.program_id(0)
    k = pl.program_id(1)
    nk = pl.num_programs(1)

    @pl.when(k == 0)
    def _():
        acc_ref[...] = jnp.zeros_like(acc_ref)

    xk = xp_ref[pl.ds(k * BK, BK), :].astype(jnp.bfloat16)

    @pl.when(k < nk - 1)
    def _():
        acc_ref[...] += jnp.dot(a_ref[...].astype(jnp.bfloat16), xk,
                                preferred_element_type=jnp.float32)

    @pl.when(k == nk - 1)
    def _():
        col = k * BK + jax.lax.broadcasted_iota(jnp.int32, (1, BK), 1)
        a = jnp.where(col < n, a_ref[...], 0.0).astype(jnp.bfloat16)
        acc = acc_ref[...] + jnp.dot(a, xk,
                                     preferred_element_type=jnp.float32)
        xr = xp_ref[pl.ds(i * BM, BM), :]
        inv = 1.0 / deg_ref[...]
        pool = inv * (acc + xr) + xr
        out = jnp.dot(pool, wt_ref[...], preferred_element_type=jnp.float32)
        out_ref[...] = jnp.maximum(out + b_ref[...], 0.0)


@jax.jit
def kernel(input_tensor, adjacency_matrix, node_degree, W, b):
    n, d_in = input_tensor.shape
    d_out = W.shape[0]
    gm = n // BM
    gk = pl.cdiv(n, BK)
    padn = gk * BK

    xp = jnp.pad(input_tensor, ((0, padn - n), (0, 0)))
    wt = W.T
    b2 = b.reshape(1, d_out)

    return pl.pallas_call(
        functools.partial(_gcn_body, n),
        grid=(gm, gk),
        in_specs=[
            pl.BlockSpec((BM, BK), lambda i, k: (i, k)),
            pl.BlockSpec((padn, d_in), lambda i, k: (0, 0)),
            pl.BlockSpec((BM, 1), lambda i, k: (i, 0)),
            pl.BlockSpec((d_in, d_out), lambda i, k: (0, 0)),
            pl.BlockSpec((1, d_out), lambda i, k: (0, 0)),
        ],
        out_specs=pl.BlockSpec((BM, d_out), lambda i, k: (i, 0)),
        out_shape=jax.ShapeDtypeStruct((n, d_out), jnp.float32),
        scratch_shapes=[pltpu.VMEM((BM, d_out), jnp.float32)],
        compiler_params=pltpu.CompilerParams(
            dimension_semantics=("parallel", "arbitrary")),
    )(adjacency_matrix, xp, node_degree, wt, b2)


# speedup vs baseline: 6.5236x; 1.0006x over previous
import jax
import jax.numpy as jnp
from jax.experimental import pallas as pl
from jax.experimental.pallas import tpu as pltpu

BM = 200


def _gcn_body(a_ref, x_ref, deg_ref, wt_ref, b_ref, out_ref):
    i = pl.program_id(0)
    acc = jnp.dot(a_ref[...].astype(jnp.bfloat16),
                  x_ref[...].astype(jnp.bfloat16),
                  preferred_element_type=jnp.float32)
    xr = x_ref[pl.ds(i * BM, BM), :]
    inv = 1.0 / deg_ref[...]
    pool = inv * (acc + xr) + xr
    out = jnp.dot(pool, wt_ref[...], preferred_element_type=jnp.float32)
    out_ref[...] = jnp.maximum(out + b_ref[...], 0.0)


@jax.jit
def kernel(input_tensor, adjacency_matrix, node_degree, W, b):
    n, d_in = input_tensor.shape
    d_out = W.shape[0]
    wt = W.T
    b2 = b.reshape(1, d_out)

    return pl.pallas_call(
        _gcn_body,
        grid=(n // BM,),
        in_specs=[
            pl.BlockSpec((BM, n), lambda i: (i, 0)),
            pl.BlockSpec((n, d_in), lambda i: (0, 0)),
            pl.BlockSpec((BM, 1), lambda i: (i, 0)),
            pl.BlockSpec((d_in, d_out), lambda i: (0, 0)),
            pl.BlockSpec((1, d_out), lambda i: (0, 0)),
        ],
        out_specs=pl.BlockSpec((BM, d_out), lambda i: (i, 0)),
        out_shape=jax.ShapeDtypeStruct((n, d_out), jnp.float32),
        compiler_params=pltpu.CompilerParams(
            dimension_semantics=("parallel",)),
    )(adjacency_matrix, input_tensor, node_degree, wt, b2)


# speedup vs baseline: 6.5975x; 1.0113x over previous
import jax
import jax.numpy as jnp
from jax.experimental import pallas as pl
from jax.experimental.pallas import tpu as pltpu

BM = 400


def _gcn_body(a_ref, x_ref, deg_ref, wt_ref, b_ref, out_ref):
    i = pl.program_id(0)
    acc = jnp.dot(a_ref[...].astype(jnp.bfloat16),
                  x_ref[...].astype(jnp.bfloat16),
                  preferred_element_type=jnp.float32)
    xr = x_ref[pl.ds(i * BM, BM), :]
    inv = 1.0 / deg_ref[...]
    pool = inv * (acc + xr) + xr
    out = jnp.dot(pool, wt_ref[...], preferred_element_type=jnp.float32)
    out_ref[...] = jnp.maximum(out + b_ref[...], 0.0)


@jax.jit
def kernel(input_tensor, adjacency_matrix, node_degree, W, b):
    n, d_in = input_tensor.shape
    d_out = W.shape[0]
    wt = W.T
    b2 = b.reshape(1, d_out)

    return pl.pallas_call(
        _gcn_body,
        grid=(n // BM,),
        in_specs=[
            pl.BlockSpec((BM, n), lambda i: (i, 0)),
            pl.BlockSpec((n, d_in), lambda i: (0, 0)),
            pl.BlockSpec((BM, 1), lambda i: (i, 0)),
            pl.BlockSpec((d_in, d_out), lambda i: (0, 0)),
            pl.BlockSpec((1, d_out), lambda i: (0, 0)),
        ],
        out_specs=pl.BlockSpec((BM, d_out), lambda i: (i, 0)),
        out_shape=jax.ShapeDtypeStruct((n, d_out), jnp.float32),
        compiler_params=pltpu.CompilerParams(
            dimension_semantics=("parallel",)),
    )(adjacency_matrix, input_tensor, node_degree, wt, b2)


# speedup vs baseline: 6.7329x; 1.0205x over previous
import jax
import jax.numpy as jnp
from jax.experimental import pallas as pl
from jax.experimental.pallas import tpu as pltpu

BM = 200


def _gcn_body(a0_ref, a1_ref, x_ref, deg_ref, wt_ref, b_ref, out_ref):
    i = pl.program_id(0)
    xb = x_ref[...].astype(jnp.bfloat16)
    acc0 = jnp.dot(a0_ref[...].astype(jnp.bfloat16), xb,
                   preferred_element_type=jnp.float32)
    acc1 = jnp.dot(a1_ref[...].astype(jnp.bfloat16), xb,
                   preferred_element_type=jnp.float32)
    acc = jnp.concatenate([acc0, acc1], axis=0)
    xr = x_ref[pl.ds(i * (2 * BM), 2 * BM), :]
    inv = 1.0 / deg_ref[...]
    pool = inv * (acc + xr) + xr
    out = jnp.dot(pool, wt_ref[...], preferred_element_type=jnp.float32)
    out_ref[...] = jnp.maximum(out + b_ref[...], 0.0)


@jax.jit
def kernel(input_tensor, adjacency_matrix, node_degree, W, b):
    n, d_in = input_tensor.shape
    d_out = W.shape[0]
    wt = W.T
    b2 = b.reshape(1, d_out)

    return pl.pallas_call(
        _gcn_body,
        grid=(n // (2 * BM),),
        in_specs=[
            pl.BlockSpec((BM, n), lambda i: (2 * i, 0)),
            pl.BlockSpec((BM, n), lambda i: (2 * i + 1, 0)),
            pl.BlockSpec((n, d_in), lambda i: (0, 0)),
            pl.BlockSpec((2 * BM, 1), lambda i: (i, 0)),
            pl.BlockSpec((d_in, d_out), lambda i: (0, 0)),
            pl.BlockSpec((1, d_out), lambda i: (0, 0)),
        ],
        out_specs=pl.BlockSpec((2 * BM, d_out), lambda i: (i, 0)),
        out_shape=jax.ShapeDtypeStruct((n, d_out), jnp.float32),
        compiler_params=pltpu.CompilerParams(
            dimension_semantics=("parallel",)),
    )(adjacency_matrix, adjacency_matrix, input_tensor, node_degree, wt, b2)
